# baseline (speedup 1.0000x reference)
import jax
import jax.numpy as jnp
from jax import lax
from jax.experimental import pallas as pl
from jax.experimental.pallas import tpu as pltpu

R = 16384
C = 1000
K = int(R * 0.7)
BC = 200
NC = C // BC


def _ohem_kernel(predt_ref, tgt_ref, out_ref, m_sc, s_sc, p_sc):
    i = pl.program_id(0)
    x = predt_ref[...]
    m_blk = jnp.max(x, axis=0)[None, :]

    @pl.when(i == 0)
    def _():
        m_sc[...] = m_blk
        s_sc[...] = jnp.zeros_like(s_sc)
        p_sc[...] = jnp.zeros_like(p_sc)

    m_old = m_sc[...]
    m_new = jnp.maximum(m_old, m_blk)
    e = jnp.exp(x - m_new)
    s_blk = lax.dot_general(
        jnp.ones((1, BC), jnp.float32), e,
        (((1,), (0,)), ((), ())),
        preferred_element_type=jnp.float32,
    )
    s_sc[...] = s_sc[...] * jnp.exp(m_old - m_new) + s_blk
    m_sc[...] = m_new

    tgt = tgt_ref[...]
    row = lax.broadcasted_iota(jnp.int32, (BC, R), 0) + i * BC
    p_sc[...] += jnp.sum(jnp.where(row == tgt, x, 0.0), axis=0)[None, :]

    @pl.when(i == NC - 1)
    def _():
        vals = m_sc[...] + jnp.log(s_sc[...]) - p_sc[...]
        u = lax.bitcast_convert_type(vals, jnp.uint32)
        sk = u ^ jnp.where(
            u >= jnp.uint32(0x80000000),
            jnp.uint32(0xFFFFFFFF),
            jnp.uint32(0x80000000),
        )

        def body(it, p):
            cand = p | (jnp.uint32(1) << (31 - it).astype(jnp.uint32))
            cnt = jnp.sum((sk >= cand).astype(jnp.int32))
            return jnp.where(cnt >= K, cand, p)

        p = lax.fori_loop(0, 32, body, jnp.uint32(0))

        gt = sk > p
        cnt_gt = jnp.sum(gt.astype(jnp.int32))
        sum_gt = jnp.sum(jnp.where(gt, vals, 0.0))
        orig = jnp.where(
            (p & jnp.uint32(0x80000000)) != jnp.uint32(0),
            p ^ jnp.uint32(0x80000000),
            ~p,
        )
        tau = lax.bitcast_convert_type(orig, jnp.float32)
        total = sum_gt + (K - cnt_gt).astype(jnp.float32) * tau
        out_ref[0, 0] = total / K


def kernel(pred, target):
    predt = pred.T
    tgt = target.astype(jnp.int32).reshape(1, R)
    out = pl.pallas_call(
        _ohem_kernel,
        grid=(NC,),
        in_specs=[
            pl.BlockSpec((BC, R), lambda i: (i, 0)),
            pl.BlockSpec((1, R), lambda i: (0, 0)),
        ],
        out_specs=pl.BlockSpec(
            (1, 1), lambda i: (0, 0), memory_space=pltpu.SMEM
        ),
        out_shape=jax.ShapeDtypeStruct((1, 1), jnp.float32),
        scratch_shapes=[
            pltpu.VMEM((1, R), jnp.float32),
            pltpu.VMEM((1, R), jnp.float32),
            pltpu.VMEM((1, R), jnp.float32),
        ],
    )(predt, tgt)
    return out[0, 0]


# speedup vs baseline: 1.0180x; 1.0180x over previous
import jax
import jax.numpy as jnp
from jax import lax
from jax.experimental import pallas as pl
from jax.experimental.pallas import tpu as pltpu

R = 16384
C = 1000
K = int(R * 0.7)
BC = 200
NC = C // BC


def _ohem_kernel(predt_ref, tgt_ref, out_ref, m_sc, s_sc, p_sc):
    i = pl.program_id(0)
    x = predt_ref[...]
    m_blk = jnp.max(x, axis=0)[None, :]

    @pl.when(i == 0)
    def _():
        m_sc[...] = m_blk
        s_sc[...] = jnp.zeros_like(s_sc)
        p_sc[...] = jnp.zeros_like(p_sc)

    m_old = m_sc[...]
    m_new = jnp.maximum(m_old, m_blk)
    e = jnp.exp(x - m_new)
    s_blk = lax.dot_general(
        jnp.ones((1, BC), jnp.float32), e,
        (((1,), (0,)), ((), ())),
        preferred_element_type=jnp.float32,
    )
    s_sc[...] = s_sc[...] * jnp.exp(m_old - m_new) + s_blk
    m_sc[...] = m_new

    tgt = tgt_ref[...] - i * BC
    row = lax.broadcasted_iota(jnp.int32, (BC, R), 0)
    p_sc[...] += jnp.sum(jnp.where(row == tgt, x, 0.0), axis=0)[None, :]

    @pl.when(i == NC - 1)
    def _():
        vals = m_sc[...] + jnp.log(s_sc[...]) - p_sc[...]
        u = lax.bitcast_convert_type(vals, jnp.uint32)
        sk = u ^ jnp.where(
            u >= jnp.uint32(0x80000000),
            jnp.uint32(0xFFFFFFFF),
            jnp.uint32(0x80000000),
        )

        def body(it, p):
            cand = p | (jnp.uint32(1) << (31 - it).astype(jnp.uint32))
            cnt = jnp.sum((sk >= cand).astype(jnp.int32))
            return jnp.where(cnt >= K, cand, p)

        p = lax.fori_loop(0, 32, body, jnp.uint32(0))

        gt = sk > p
        cnt_gt = jnp.sum(gt.astype(jnp.int32))
        sum_gt = jnp.sum(jnp.where(gt, vals, 0.0))
        orig = jnp.where(
            (p & jnp.uint32(0x80000000)) != jnp.uint32(0),
            p ^ jnp.uint32(0x80000000),
            ~p,
        )
        tau = lax.bitcast_convert_type(orig, jnp.float32)
        total = sum_gt + (K - cnt_gt).astype(jnp.float32) * tau
        out_ref[0, 0] = total / K


def kernel(pred, target):
    predt = pred.T
    tgt = target.astype(jnp.int32).reshape(1, R)
    out = pl.pallas_call(
        _ohem_kernel,
        grid=(NC,),
        in_specs=[
            pl.BlockSpec((BC, R), lambda i: (i, 0)),
            pl.BlockSpec((1, R), lambda i: (0, 0)),
        ],
        out_specs=pl.BlockSpec(
            (1, 1), lambda i: (0, 0), memory_space=pltpu.SMEM
        ),
        out_shape=jax.ShapeDtypeStruct((1, 1), jnp.float32),
        scratch_shapes=[
            pltpu.VMEM((1, R), jnp.float32),
            pltpu.VMEM((1, R), jnp.float32),
            pltpu.VMEM((1, R), jnp.float32),
        ],
    )(predt, tgt)
    return out[0, 0]
